# baseline (speedup 1.0000x reference)
import functools

import jax
import jax.numpy as jnp
from jax.experimental import pallas as pl

N_HEADS, HEAD_DIM, TOPK = 8, 64, 512
EPS = 1e-6


def _rms_norm(x, w):
    xf = x.astype(jnp.float32)
    var = jnp.mean(xf * xf, axis=-1, keepdims=True)
    return (xf * jax.lax.rsqrt(var + EPS)) * w


def _rotate_half(x):
    h = x.shape[-1] // 2
    x1, x2 = x[..., :h], x[..., h:]
    return jnp.concatenate([-x2, x1], axis=-1)


def _score_kernel(q_ref, k_ref, w_ref, out_ref):
    q = q_ref[0]
    k = k_ref[0]
    w = w_ref[0]
    s, n, d = q.shape
    t = k.shape[0]
    qs = q.reshape(s * n, d)
    sc = jax.lax.dot_general(qs, k, (((1,), (1,)), ((), ())),
                             preferred_element_type=jnp.float32)
    sc = sc.reshape(s, n, t)
    sc = jax.nn.relu(sc)
    sc = (sc * w[:, :, None]).sum(axis=1)
    out_ref[0] = sc


def kernel(x, cos, sin, Wq, Wk, Ww, q_norm_w, k_norm_w, start_pos, end_pos):
    bsz, seqlen, _ = x.shape
    softmax_scale = HEAD_DIM ** (-0.5)
    q = (x @ Wq).reshape(bsz, seqlen, N_HEADS, HEAD_DIM)
    q = _rms_norm(q, q_norm_w)
    q = jnp.transpose(q, (0, 2, 1, 3))
    k = (x @ Wk).reshape(bsz, seqlen, 1, HEAD_DIM)
    k = _rms_norm(k, k_norm_w)
    k = jnp.transpose(k, (0, 2, 1, 3))
    cos_e = cos[:, None, :, :]
    sin_e = sin[:, None, :, :]
    q = q * cos_e + _rotate_half(q) * sin_e
    k = k * cos_e + _rotate_half(k) * sin_e
    q = jnp.transpose(q, (0, 2, 1, 3))
    k = jnp.transpose(k, (0, 2, 1, 3))[:, :, 0, :]
    weights = (x @ Ww) * (N_HEADS ** (-0.5)) * softmax_scale

    SB = 256
    score = pl.pallas_call(
        _score_kernel,
        grid=(bsz, seqlen // SB),
        in_specs=[
            pl.BlockSpec((1, SB, N_HEADS, HEAD_DIM), lambda b, i: (b, i, 0, 0)),
            pl.BlockSpec((1, seqlen, HEAD_DIM), lambda b, i: (b, 0, 0)),
            pl.BlockSpec((1, SB, N_HEADS), lambda b, i: (b, i, 0)),
        ],
        out_specs=pl.BlockSpec((1, SB, seqlen), lambda b, i: (b, i, 0)),
        out_shape=jax.ShapeDtypeStruct((bsz, seqlen, seqlen), jnp.float32),
    )(q, k, weights)

    mask = jnp.triu(jnp.full((seqlen, seqlen), -jnp.inf, dtype=score.dtype), 1)
    score = score + mask
    topk_indices = jax.lax.top_k(score, min(TOPK, seqlen))[1]
    return topk_indices, score
